# Initial kernel scaffold; baseline (speedup 1.0000x reference)
import jax
import jax.numpy as jnp
from jax.experimental import pallas as pl


def kernel(input_images, indices, mask):
    raise NotImplementedError("write your pallas kernel here")


# speedup vs baseline: 1.2491x; 1.2491x over previous
import functools

import jax
import jax.numpy as jnp
from jax import lax
from jax.experimental import pallas as pl
from jax.experimental.pallas import tpu as pltpu
from jax.experimental.pallas import tpu_sc as plsc

B, F, IMG = 4, 384, 128 * 128
L, K = 64 * 64, 9
BF = B * F

NC, NS, LANES = 2, 16, 16
NW = NC * NS
ROWS_PER_W = BF // NW
G = 2
NGRP = ROWS_PER_W // G
ROW_PAD = IMG + LANES
NCHUNK = L // LANES

_mesh = plsc.VectorSubcoreMesh(core_axis_name="c", subcore_axis_name="s")


@functools.partial(
    pl.kernel,
    mesh=_mesh,
    compiler_params=pltpu.CompilerParams(needs_layout_passes=False),
    out_type=jax.ShapeDtypeStruct((BF * L,), jnp.float32),
    scratch_types=[
        pltpu.VMEM((K * L,), jnp.int32),
        pltpu.VMEM((K * L,), jnp.float32),
        pltpu.VMEM((G * ROW_PAD,), jnp.float32),
        pltpu.VMEM((G * L,), jnp.float32),
    ],
)
def _pool(x_hbm, idx_hbm, m_hbm, out_hbm, cidx_v, m_v, rows_v, out_v):
    wid = lax.axis_index("s") * NC + lax.axis_index("c")
    row0 = wid * ROWS_PER_W

    pltpu.sync_copy(idx_hbm, cidx_v)
    pltpu.sync_copy(m_hbm, m_v)

    def fix_body(i, carry):
        off = pl.multiple_of(i * LANES, LANES)
        idx = cidx_v[pl.ds(off, LANES)]
        m = m_v[pl.ds(off, LANES)]
        cidx_v[pl.ds(off, LANES)] = jnp.where(m > 0.0, idx, IMG)
        return carry

    lax.fori_loop(0, (K * L) // LANES, fix_body, 0)

    for r in range(G):
        rows_v[pl.ds(r * ROW_PAD + IMG, LANES)] = jnp.zeros((LANES,), jnp.float32)

    inv_k = jnp.float32(1.0 / K)

    def group_body(g, carry):
        base = row0 + g * G
        for r in range(G):
            pltpu.sync_copy(
                x_hbm.at[pl.ds((base + r) * IMG, IMG)],
                rows_v.at[pl.ds(r * ROW_PAD, IMG)],
            )

        def chunk_body(c, inner):
            off = pl.multiple_of(c * LANES, LANES)
            ci = [cidx_v[pl.ds(k * L + off, LANES)] for k in range(K)]
            for r in range(G):
                cr = [x + (r * ROW_PAD) for x in ci] if r else ci
                acc = plsc.load_gather(rows_v, [cr[0]])
                for k in range(1, K):
                    acc = acc + plsc.load_gather(rows_v, [cr[k]])
                out_v[pl.ds(r * L + off, LANES)] = acc * inv_k
            return inner

        lax.fori_loop(0, NCHUNK, chunk_body, 0)
        pltpu.sync_copy(out_v, out_hbm.at[pl.ds(base * L, G * L)])
        return carry

    lax.fori_loop(0, NGRP, group_body, 0)


def kernel(input_images, indices, mask):
    x = input_images.reshape(BF * IMG)
    out = _pool(x, indices.reshape(K * L), mask.reshape(K * L))
    return out.reshape(B, F, L)
